# baseline (speedup 1.0000x reference)
import functools

import jax
import jax.numpy as jnp
from jax import lax
from jax.experimental import pallas as pl
from jax.experimental.pallas import tpu as pltpu
from jax.experimental.pallas import tpu_sc as plsc

N = 10000
E = 640000
DIM = 100
DAG = 50
H = 128
D = 64
C2 = 256
NP = 10240
ZROW = 10008

NC = 2
NS = 16
NW = NC * NS
R = NP // NW
HALF = NP // NC
RT = HALF // NS
SUB = 5
MAXRANK = 256

RB = 2560
GRID = NP // RB


def _pad2(a, r, c):
    return jnp.zeros((r, c), jnp.float32).at[: a.shape[0], : a.shape[1]].set(a)


def _padb(b, w=H):
    return jnp.zeros((1, w), jnp.float32).at[0, : b.shape[0]].set(b)


def _init_body(feat_ref, evt_ref, edt_ref, w1_ref, b1_ref, w2_ref, b2_ref,
               o_ref):
    f0 = feat_ref[:, 0:1]
    f1 = feat_ref[:, 1:2]
    ev = jnp.where(f0 == 0, evt_ref[0:1], 0.0)
    for k in range(1, 4):
        ev = ev + jnp.where(f0 == k, evt_ref[k:k + 1], 0.0)
    ed = jnp.where(f1 == 0, edt_ref[0:1], 0.0)
    for k in range(1, 3):
        ed = ed + jnp.where(f1 == k, edt_ref[k:k + 1], 0.0)
    x0 = jnp.concatenate([ev, ed], axis=1)
    t = jnp.maximum(jnp.dot(x0, w1_ref[...],
                            preferred_element_type=jnp.float32) + b1_ref[...],
                    0.0)
    o_ref[...] = jnp.dot(t, w2_ref[...],
                         preferred_element_type=jnp.float32) + b2_ref[...]


def _init_call(feat, evt, edt, w1, b1, w2, b2):
    return pl.pallas_call(
        _init_body,
        grid=(GRID,),
        in_specs=[
            pl.BlockSpec((RB, 2), lambda i: (i, 0)),
            pl.BlockSpec((8, H), lambda i: (0, 0)),
            pl.BlockSpec((8, H), lambda i: (0, 0)),
            pl.BlockSpec((C2, H), lambda i: (0, 0)),
            pl.BlockSpec((1, H), lambda i: (0, 0)),
            pl.BlockSpec((H, H), lambda i: (0, 0)),
            pl.BlockSpec((1, H), lambda i: (0, 0)),
        ],
        out_specs=pl.BlockSpec((RB, H), lambda i: (i, 0)),
        out_shape=jax.ShapeDtypeStruct((NP, H), jnp.float32),
    )(feat, evt, edt, w1, b1, w2, b2)


def _mlp_body(h_ref, w1_ref, b1_ref, w2_ref, b2_ref, o_ref):
    t = jnp.maximum(jnp.dot(h_ref[...], w1_ref[...],
                            preferred_element_type=jnp.float32) + b1_ref[...],
                    0.0)
    o = jnp.dot(t, w2_ref[...],
                preferred_element_type=jnp.float32) + b2_ref[...]
    rid = lax.broadcasted_iota(jnp.int32, (RB, 1), 0) + pl.program_id(0) * RB
    o_ref[...] = jnp.where(rid < N, o, 0.0)


def _mlp_call(h, w1, b1, w2, b2):
    return pl.pallas_call(
        _mlp_body,
        grid=(GRID,),
        in_specs=[
            pl.BlockSpec((RB, H), lambda i: (i, 0)),
            pl.BlockSpec((H, D), lambda i: (0, 0)),
            pl.BlockSpec((1, D), lambda i: (0, 0)),
            pl.BlockSpec((D, H), lambda i: (0, 0)),
            pl.BlockSpec((1, H), lambda i: (0, 0)),
        ],
        out_specs=pl.BlockSpec((RB, H), lambda i: (i, 0)),
        out_shape=jax.ShapeDtypeStruct((NP, H), jnp.float32),
    )(h, w1, b1, w2, b2)


def _gru_body(y_ref, h_ref, ri_ref, zi_ref, ni_ref,
              rh_ref, zh_ref, nh_ref, bi_ref, bh_ref, o_ref):
    x = y_ref[...]
    h = h_ref[...]
    dot = lambda a, b: jnp.dot(a, b, preferred_element_type=jnp.float32)
    ir = dot(x, ri_ref[...]) + bi_ref[0:1]
    iz = dot(x, zi_ref[...]) + bi_ref[1:2]
    inn = dot(x, ni_ref[...]) + bi_ref[2:3]
    hr = dot(h, rh_ref[...]) + bh_ref[0:1]
    hz = dot(h, zh_ref[...]) + bh_ref[1:2]
    hn = dot(h, nh_ref[...]) + bh_ref[2:3]
    r = jax.nn.sigmoid(ir + hr)
    z = jax.nn.sigmoid(iz + hz)
    n = jnp.tanh(inn + r * hn)
    o_ref[...] = (1.0 - z) * n + z * h


def _gru_call(y, h, ri, zi, ni, rh, zh, nh, bi3, bh3):
    return pl.pallas_call(
        _gru_body,
        grid=(GRID,),
        in_specs=[
            pl.BlockSpec((RB, H), lambda i: (i, 0)),
            pl.BlockSpec((RB, H), lambda i: (i, 0)),
        ] + [pl.BlockSpec((H, H), lambda i: (0, 0))] * 6 + [
            pl.BlockSpec((3, H), lambda i: (0, 0)),
            pl.BlockSpec((3, H), lambda i: (0, 0)),
        ],
        out_specs=pl.BlockSpec((RB, H), lambda i: (i, 0)),
        out_shape=jax.ShapeDtypeStruct((NP, H), jnp.float32),
    )(y, h, ri, zi, ni, rh, zh, nh, bi3, bh3)


def _spmm_body(x_hbm, lay_hbm, cnt_hbm, iota_hbm, zeros_hbm, out_hbm,
               gi_v, io_v, cnt_v, rows_v, acc_sh, sem):
    c = lax.axis_index("c")
    s = lax.axis_index("s")
    wid = c * NS + s

    pltpu.sync_copy(cnt_hbm, cnt_v)
    pltpu.sync_copy(iota_hbm.at[wid], io_v)

    pltpu.sync_copy(zeros_hbm.at[pl.ds(s * RT, RT)],
                    acc_sh.at[pl.ds(s * RT, RT)])
    plsc.subcore_barrier()

    def chunk(r, carry):
        pltpu.sync_copy(lay_hbm.at[wid, r], gi_v)
        descs = []
        for j in range(SUB):
            descs.append(pltpu.async_copy(
                x_hbm.at[gi_v.at[j]],
                rows_v.at[pl.ds(j * 64, 64)], sem))
        for d in descs:
            d.wait()
        for j in range(SUB):
            pltpu.sync_copy(rows_v.at[pl.ds(j * 64, 64)],
                            acc_sh.at[io_v.at[j]], add=True)
        return carry

    cnt = cnt_v[pl.ds(wid, 16)][0]
    lax.fori_loop(0, cnt, chunk, 0)
    plsc.subcore_barrier()
    pltpu.sync_copy(acc_sh.at[pl.ds(s * RT, RT)],
                    out_hbm.at[pl.ds(c * HALF + s * RT, RT)])


@functools.cache
def _spmm_kernel():
    return pl.kernel(
        _spmm_body,
        out_type=jax.ShapeDtypeStruct((NP, H), jnp.float32),
        mesh=plsc.VectorSubcoreMesh(core_axis_name="c", subcore_axis_name="s",
                                    num_cores=NC, num_subcores=NS),
        scratch_types=[
            pltpu.VMEM((SUB, 64), jnp.int32),
            pltpu.VMEM((SUB, 64), jnp.int32),
            pltpu.VMEM((NW + 16,), jnp.int32),
            pltpu.VMEM((R, H), jnp.float32),
            pltpu.VMEM_SHARED((HALF, H), jnp.float32),
            pltpu.SemaphoreType.DMA,
        ],
        compiler_params=pltpu.CompilerParams(use_tc_tiling_on_sc=False),
    )


def _spmm(x, lay, cnt, iota3, zeros_nd):
    return _spmm_kernel()(x, lay, cnt, iota3, zeros_nd)


def _build_layout(dst, src):
    order = jnp.argsort(dst, stable=True)
    ds = dst[order]
    ss = src[order]
    deg = jnp.zeros((NP,), jnp.int32).at[ds].add(1)
    starts = jnp.concatenate(
        [jnp.zeros((1,), jnp.int32), jnp.cumsum(deg)[:-1].astype(jnp.int32)])
    rank = jnp.arange(E, dtype=jnp.int32) - starts[ds]
    w = ds // R
    flat = (w * MAXRANK + rank) * R + (ds % R)
    flat = jnp.where(rank < MAXRANK, flat, NW * MAXRANK * R)
    lay = (jnp.full((NW * MAXRANK * R,), ZROW, jnp.int32)
           .at[flat].set(ss, mode="drop"))
    cnt = jnp.minimum(
        jnp.zeros((NW,), jnp.int32).at[w].max(rank + 1), MAXRANK)
    cnt = jnp.concatenate([cnt, jnp.zeros((16,), jnp.int32)])
    return lay.reshape(NW, MAXRANK, SUB, 64), cnt


def kernel(edge_index, features, params):
    p = params
    row = edge_index[0].astype(jnp.int32)
    col = edge_index[1].astype(jnp.int32)

    lay_f, cnt_f = _build_layout(row, col)
    lay_b, cnt_b = _build_layout(col, row)
    iota3 = ((jnp.arange(NW, dtype=jnp.int32) % NS)[:, None] * R
             + jnp.arange(R, dtype=jnp.int32)[None, :]).reshape(NW, SUB, 64)

    feat = jnp.zeros((NP, 2), jnp.int32).at[:N].set(features)
    evt = _pad2(p["emb_var"], 8, H)
    edt = _pad2(p["emb_dec"], 8, H)

    Wi1p = (jnp.zeros((C2, H), jnp.float32)
            .at[:DIM, :DIM].set(p["Wi1"][:DIM])
            .at[H:H + DIM, :DIM].set(p["Wi1"][DIM:]))
    bi1p = _padb(p["bi1"])
    Wi2p = _pad2(p["Wi2"], H, H)
    bi2p = _padb(p["bi2"])

    Wf1p = _pad2(p["Wf1"], H, D)
    bf1p = jnp.zeros((1, D), jnp.float32).at[0, :DAG].set(p["bf1"])
    Wf2p = _pad2(p["Wf2"], D, H)
    bf2p = _padb(p["bf2"])
    Wb1p = _pad2(p["Wb1"], H, D)
    bb1p = jnp.zeros((1, D), jnp.float32).at[0, :DAG].set(p["bb1"])
    Wb2p = _pad2(p["Wb2"], D, H)
    bb2p = _padb(p["bb2"])

    def gru_params(wih, whh, bih, bhh):
        ws = [_pad2(wih[k * DIM:(k + 1) * DIM].T, H, H) for k in range(3)]
        hs = [_pad2(whh[k * DIM:(k + 1) * DIM].T, H, H) for k in range(3)]
        bi3 = jnp.stack([_padb(bih[k * DIM:(k + 1) * DIM])[0]
                         for k in range(3)])
        bh3 = jnp.stack([_padb(bhh[k * DIM:(k + 1) * DIM])[0]
                         for k in range(3)])
        return ws + hs + [bi3, bh3]

    gf = gru_params(p["Wih_f"], p["Whh_f"], p["bih_f"], p["bhh_f"])
    gb = gru_params(p["Wih_b"], p["Whh_b"], p["bih_b"], p["bhh_b"])

    Wc1p = _pad2(p["Wc1"], H, D)
    bc1p = jnp.zeros((1, D), jnp.float32).at[0, :30].set(p["bc1"])
    Wc2p = _pad2(p["Wc2"], D, H)
    bc2p = _padb(p["bc2"])

    zeros_nd = jnp.zeros((HALF, H), jnp.float32)

    h = _init_call(feat, evt, edt, Wi1p, bi1p, Wi2p, bi2p)

    def round_body(_, h):
        x = _mlp_call(h, Wf1p, bf1p, Wf2p, bf2p)
        y = _spmm(x, lay_f, cnt_f, iota3, zeros_nd)
        h = _gru_call(y, h, *gf)
        x = _mlp_call(h, Wb1p, bb1p, Wb2p, bb2p)
        y = _spmm(x, lay_b, cnt_b, iota3, zeros_nd)
        h = _gru_call(y, h, *gb)
        return h

    h = lax.fori_loop(0, 20, round_body, h)
    out = _mlp_call(h, Wc1p, bc1p, Wc2p, bc2p)
    return out[:N, :1]
